# Initial kernel scaffold; baseline (speedup 1.0000x reference)
import jax
import jax.numpy as jnp
from jax.experimental import pallas as pl


def kernel(input, mask, weight, bias):
    raise NotImplementedError("write your pallas kernel here")


# speedup vs baseline: 1.7238x; 1.7238x over previous
import functools

import jax
import jax.numpy as jnp
from jax import lax
from jax.experimental import pallas as pl
from jax.experimental.pallas import tpu as pltpu
from jax.experimental.pallas import tpu_sc as plsc

B = 2
C = 256
K = 184
H = 224
W = 224
P = H * W

NW = 32
CHUNK = 256
NCH = (B * P) // CHUNK
CPB = P // CHUNK
CPW = -(-NCH // NW)
NGRP = CHUNK // 16

TCBLK = 512
NBLK = P // TCBLK


def _sc_argmax_body(mask_hbm, idx_hbm, buf, idxc, sem):
    wid = lax.axis_index("s") * 2 + lax.axis_index("c")
    for t in range(CPW):
        cid = t * NW + wid

        @pl.when(cid < NCH)
        def _():
            b = cid // CPB
            off = pl.multiple_of((cid % CPB) * CHUNK, CHUNK)
            pltpu.async_copy(
                mask_hbm.at[b, :, pl.ds(off, CHUNK)], buf, sem
            ).wait()
            for g in range(NGRP):
                sl = pl.ds(g * 16, 16)

                def body(ch, carry, sl=sl):
                    maxv, idxv = carry
                    v = buf[ch, sl]
                    pred = v > maxv
                    return (
                        jnp.where(pred, v, maxv),
                        jnp.where(pred, jnp.full((16,), ch, jnp.int32), idxv),
                    )

                maxv0 = buf[0, sl]
                _, idxv = lax.fori_loop(
                    1, K, body, (maxv0, jnp.zeros((16,), jnp.int32))
                )
                idxc[sl] = idxv
            pltpu.sync_copy(
                idxc, idx_hbm.at[pl.ds(pl.multiple_of(cid * CHUNK, CHUNK), CHUNK)]
            )


def _sc_argmax(mask3):
    return pl.kernel(
        _sc_argmax_body,
        out_type=jax.ShapeDtypeStruct((B * P,), jnp.int32),
        mesh=plsc.VectorSubcoreMesh(core_axis_name="c", subcore_axis_name="s"),
        scratch_types=[
            pltpu.VMEM((K, CHUNK), jnp.float32),
            pltpu.VMEM((CHUNK,), jnp.int32),
            pltpu.SemaphoreType.DMA,
        ],
    )(mask3)


def _tc_affine_body(wt_ref, bt_ref, idx_ref, in_ref, out_ref):
    idx = idx_ref[0, 0, :]
    iot = lax.broadcasted_iota(jnp.int32, (K, TCBLK), 0)
    onehot = (iot == idx[None, :]).astype(jnp.float32)
    cw = jnp.dot(wt_ref[...], onehot, preferred_element_type=jnp.float32)
    cb = jnp.dot(bt_ref[...], onehot, preferred_element_type=jnp.float32)
    out_ref[0] = in_ref[0] * cw + cb


def _tc_affine(wt, bt, idx3, inp3):
    return pl.pallas_call(
        _tc_affine_body,
        grid=(B, NBLK),
        in_specs=[
            pl.BlockSpec((C, K), lambda b, j: (0, 0)),
            pl.BlockSpec((C, K), lambda b, j: (0, 0)),
            pl.BlockSpec((1, 1, TCBLK), lambda b, j: (b, 0, j)),
            pl.BlockSpec((1, C, TCBLK), lambda b, j: (b, 0, j)),
        ],
        out_specs=pl.BlockSpec((1, C, TCBLK), lambda b, j: (b, 0, j)),
        out_shape=jax.ShapeDtypeStruct((B, C, P), jnp.float32),
    )(wt, bt, idx3, inp3)


def kernel(input, mask, weight, bias):
    mask3 = mask.reshape(B, K, P)
    idx = _sc_argmax(mask3)
    idx3 = idx.reshape(B, 1, P)
    out = _tc_affine(weight.T, bias.T, idx3, input.reshape(B, C, P))
    return out.reshape(B, C, H, W)


# speedup vs baseline: 2.0171x; 1.1701x over previous
import functools

import jax
import jax.numpy as jnp
from jax import lax
from jax.experimental import pallas as pl
from jax.experimental.pallas import tpu as pltpu
from jax.experimental.pallas import tpu_sc as plsc

B = 2
C = 256
K = 184
H = 224
W = 224
P = H * W

NW = 32
CHUNK = 256
NCH = (B * P) // CHUNK
CPB = P // CHUNK
CPW = -(-NCH // NW)
NGRP = CHUNK // 16

TCBLK = 1792
NBLK = P // TCBLK


def _sc_argmax_body(mask_hbm, idx_hbm, buf, idxc, sem):
    wid = lax.axis_index("s") * 2 + lax.axis_index("c")
    for t in range(CPW):
        cid = t * NW + wid

        @pl.when(cid < NCH)
        def _():
            b = cid // CPB
            off = pl.multiple_of((cid % CPB) * CHUNK, CHUNK)
            pltpu.async_copy(
                mask_hbm.at[b, :, pl.ds(off, CHUNK)], buf, sem
            ).wait()
            for g in range(NGRP):
                sl = pl.ds(g * 16, 16)

                def body(ch, carry, sl=sl):
                    maxv, idxv = carry
                    v = buf[ch, sl]
                    pred = v > maxv
                    return (
                        jnp.where(pred, v, maxv),
                        jnp.where(pred, jnp.full((16,), ch, jnp.int32), idxv),
                    )

                maxv0 = buf[0, sl]
                _, idxv = lax.fori_loop(
                    1, K, body, (maxv0, jnp.zeros((16,), jnp.int32))
                )
                idxc[sl] = idxv
            pltpu.sync_copy(
                idxc, idx_hbm.at[pl.ds(pl.multiple_of(cid * CHUNK, CHUNK), CHUNK)]
            )


def _sc_argmax(mask3):
    return pl.kernel(
        _sc_argmax_body,
        out_type=jax.ShapeDtypeStruct((B * P,), jnp.int32),
        mesh=plsc.VectorSubcoreMesh(core_axis_name="c", subcore_axis_name="s"),
        scratch_types=[
            pltpu.VMEM((K, CHUNK), jnp.float32),
            pltpu.VMEM((CHUNK,), jnp.int32),
            pltpu.SemaphoreType.DMA,
        ],
    )(mask3)


def _tc_affine_body(wb_ref, idx_ref, in_ref, out_ref):
    idx = idx_ref[0, 0, :]
    iot = lax.broadcasted_iota(jnp.int32, (K, TCBLK), 0)
    onehot = (iot == idx[None, :]).astype(jnp.bfloat16)
    cwb = jnp.dot(wb_ref[...], onehot, preferred_element_type=jnp.float32)
    out_ref[0] = in_ref[0] * cwb[:C] + cwb[C:]


def _tc_affine(wb, idx3, inp3):
    return pl.pallas_call(
        _tc_affine_body,
        grid=(B, NBLK),
        in_specs=[
            pl.BlockSpec((2 * C, K), lambda b, j: (0, 0)),
            pl.BlockSpec((1, 1, TCBLK), lambda b, j: (b, 0, j)),
            pl.BlockSpec((1, C, TCBLK), lambda b, j: (b, 0, j)),
        ],
        out_specs=pl.BlockSpec((1, C, TCBLK), lambda b, j: (b, 0, j)),
        out_shape=jax.ShapeDtypeStruct((B, C, P), jnp.float32),
    )(wb, idx3, inp3)


def kernel(input, mask, weight, bias):
    mask3 = mask.reshape(B, K, P)
    idx = _sc_argmax(mask3)
    idx3 = idx.reshape(B, 1, P)
    wb = jnp.concatenate([weight.T, bias.T]).astype(jnp.bfloat16)
    out = _tc_affine(wb, idx3, input.reshape(B, C, P))
    return out.reshape(B, C, H, W)


# speedup vs baseline: 2.4025x; 1.1911x over previous
import jax
import jax.numpy as jnp
from jax import lax
from jax.experimental import pallas as pl
from jax.experimental.pallas import tpu as pltpu
from jax.experimental.pallas import tpu_sc as plsc

B = 2
C = 256
K = 184
H = 224
W = 224

NW = 32
RB = 8
NPOS = B * (H // RB)
PPB = H // RB
KC = 23
NKC = K // KC
NG = W // 16

HB = 16
NRB = H // HB


def _sc_argmax_body(mask_hbm, idx_hbm, buf0, buf1, runmax, runidx, sem0, sem1):
    wid = lax.axis_index("s") * 2 + lax.axis_index("c")
    bufs = (buf0, buf1)
    sems = (sem0, sem1)

    def src(i):
        p, q = divmod(i, NKC)
        pos = wid + p * NW
        b = pos // PPB
        h0 = pl.multiple_of((pos % PPB) * RB, RB)
        return pos, mask_hbm.at[b, pl.ds(q * KC, KC), pl.ds(h0, RB), :]

    def compute(i):
        p, q = divmod(i, NKC)
        buf = bufs[i % 2]
        qbase = q * KC
        sls = [pl.ds(g * 16, 16) for g in range(NG)]

        def row_body(r, _):
            if q == 0:
                init = tuple(
                    (buf[0, r, sl], jnp.zeros((16,), jnp.int32)) for sl in sls
                )
                lo = 1
            else:
                init = tuple((runmax[r, sl], runidx[r, sl]) for sl in sls)
                lo = 0

            def ch_body(ch, carry):
                chv = jnp.full((16,), qbase + ch, jnp.int32)
                out = []
                for g, sl in enumerate(sls):
                    maxv, idxv = carry[g]
                    v = buf[ch, r, sl]
                    pred = v > maxv
                    out.append((jnp.where(pred, v, maxv),
                                jnp.where(pred, chv, idxv)))
                return tuple(out)

            res = lax.fori_loop(lo, KC, ch_body, init)
            for g, sl in enumerate(sls):
                if q < NKC - 1:
                    runmax[r, sl] = res[g][0]
                runidx[r, sl] = res[g][1]
            return 0

        lax.fori_loop(0, RB, row_body, 0)

        if q == NKC - 1:
            pos = wid + p * NW
            b = pos // PPB
            h0 = pl.multiple_of((pos % PPB) * RB, RB)
            pltpu.sync_copy(runidx, idx_hbm.at[b, pl.ds(h0, RB), :])

    NT = 2 * NKC

    def valid(i):
        return (wid + (i // NKC) * NW) < NPOS

    _, s0 = src(0)
    pltpu.make_async_copy(s0, bufs[0], sems[0]).start()
    for i in range(NT):
        if i + 1 < NT:
            _, sn = src(i + 1)

            @pl.when(valid(i + 1))
            def _(sn=sn, i=i):
                pltpu.make_async_copy(sn, bufs[(i + 1) % 2], sems[(i + 1) % 2]).start()

        _, s = src(i)

        @pl.when(valid(i))
        def _(s=s, i=i):
            pltpu.make_async_copy(s, bufs[i % 2], sems[i % 2]).wait()
            compute(i)


def _sc_argmax(mask):
    return pl.kernel(
        _sc_argmax_body,
        out_type=jax.ShapeDtypeStruct((B, H, W), jnp.int32),
        mesh=plsc.VectorSubcoreMesh(core_axis_name="c", subcore_axis_name="s"),
        scratch_types=[
            pltpu.VMEM((KC, RB, W), jnp.float32),
            pltpu.VMEM((KC, RB, W), jnp.float32),
            pltpu.VMEM((RB, W), jnp.float32),
            pltpu.VMEM((RB, W), jnp.int32),
            pltpu.SemaphoreType.DMA,
            pltpu.SemaphoreType.DMA,
        ],
    )(mask)


def _tc_affine_body(wb_ref, idx_ref, in_ref, out_ref):
    wbv = wb_ref[...]
    idxf = idx_ref[0, 0, :]
    for r in range(HB):
        idxr = lax.slice(idxf, (r * W,), ((r + 1) * W,))
        iot = lax.broadcasted_iota(jnp.int32, (K, W), 0)
        onehot = (iot == idxr[None, :]).astype(jnp.bfloat16)
        cwb = jnp.dot(wbv, onehot, preferred_element_type=jnp.float32)
        out_ref[0, :, r, :] = in_ref[0, :, r, :] * cwb[:C] + cwb[C:]


def _tc_affine(wb, idxf3, inp):
    return pl.pallas_call(
        _tc_affine_body,
        grid=(B, NRB),
        in_specs=[
            pl.BlockSpec((2 * C, K), lambda b, j: (0, 0)),
            pl.BlockSpec((1, 1, HB * W), lambda b, j: (b, 0, j)),
            pl.BlockSpec((1, C, HB, W), lambda b, j: (b, 0, j, 0)),
        ],
        out_specs=pl.BlockSpec((1, C, HB, W), lambda b, j: (b, 0, j, 0)),
        out_shape=jax.ShapeDtypeStruct((B, C, H, W), jnp.float32),
    )(wb, idxf3, inp)


def kernel(input, mask, weight, bias):
    idx = _sc_argmax(mask)
    wb = jnp.concatenate([weight.T, bias.T]).astype(jnp.bfloat16)
    return _tc_affine(wb, idx.reshape(B, 1, H * W), input)


# speedup vs baseline: 6.8498x; 2.8512x over previous
import jax
import jax.numpy as jnp
from jax import lax
from jax.experimental import pallas as pl
from jax.experimental.pallas import tpu as pltpu
from jax.experimental.pallas import tpu_sc as plsc

B = 2
C = 256
K = 184
H = 224
W = 224

NW = 32
RB = 8
NPOS = B * (H // RB)
PPB = H // RB
KC = 23
NKC = K // KC
NG = W // 16

P = H * W
NPX = 3584
NBLK = P // NPX


def _sc_argmax_body(mask_hbm, idx_hbm, buf0, buf1, runmax, runidx, sem0, sem1):
    wid = lax.axis_index("s") * 2 + lax.axis_index("c")
    bufs = (buf0, buf1)
    sems = (sem0, sem1)

    def src(i):
        p, q = divmod(i, NKC)
        pos = wid + p * NW
        b = pos // PPB
        h0 = pl.multiple_of((pos % PPB) * RB, RB)
        return pos, mask_hbm.at[b, pl.ds(q * KC, KC), pl.ds(h0, RB), :]

    def compute(i):
        p, q = divmod(i, NKC)
        buf = bufs[i % 2]
        qbase = q * KC
        sls = [pl.ds(g * 16, 16) for g in range(NG)]

        def row_body(r, _):
            if q == 0:
                init = tuple(
                    (buf[0, r, sl], jnp.zeros((16,), jnp.int32)) for sl in sls
                )
                lo = 1
            else:
                init = tuple((runmax[r, sl], runidx[r, sl]) for sl in sls)
                lo = 0

            def ch_body(ch, carry):
                chv = jnp.full((16,), qbase + ch, jnp.int32)
                out = []
                for g, sl in enumerate(sls):
                    maxv, idxv = carry[g]
                    v = buf[ch, r, sl]
                    pred = v > maxv
                    out.append((jnp.where(pred, v, maxv),
                                jnp.where(pred, chv, idxv)))
                return tuple(out)

            res = lax.fori_loop(lo, KC, ch_body, init)
            for g, sl in enumerate(sls):
                if q < NKC - 1:
                    runmax[r, sl] = res[g][0]
                runidx[r, sl] = res[g][1]
            return 0

        lax.fori_loop(0, RB, row_body, 0)

        if q == NKC - 1:
            pos = wid + p * NW
            b = pos // PPB
            h0 = pl.multiple_of((pos % PPB) * RB, RB)
            pltpu.sync_copy(runidx, idx_hbm.at[b, pl.ds(h0, RB), :])

    NT = 2 * NKC

    def valid(i):
        return (wid + (i // NKC) * NW) < NPOS

    _, s0 = src(0)
    pltpu.make_async_copy(s0, bufs[0], sems[0]).start()
    for i in range(NT):
        if i + 1 < NT:
            _, sn = src(i + 1)

            @pl.when(valid(i + 1))
            def _(sn=sn, i=i):
                pltpu.make_async_copy(sn, bufs[(i + 1) % 2], sems[(i + 1) % 2]).start()

        _, s = src(i)

        @pl.when(valid(i))
        def _(s=s, i=i):
            pltpu.make_async_copy(s, bufs[i % 2], sems[i % 2]).wait()
            compute(i)


def _sc_argmax(mask):
    return pl.kernel(
        _sc_argmax_body,
        out_type=jax.ShapeDtypeStruct((B, H, W), jnp.int32),
        mesh=plsc.VectorSubcoreMesh(core_axis_name="c", subcore_axis_name="s"),
        scratch_types=[
            pltpu.VMEM((KC, RB, W), jnp.float32),
            pltpu.VMEM((KC, RB, W), jnp.float32),
            pltpu.VMEM((RB, W), jnp.float32),
            pltpu.VMEM((RB, W), jnp.int32),
            pltpu.SemaphoreType.DMA,
            pltpu.SemaphoreType.DMA,
        ],
    )(mask)


def _tc_affine_body(wb_ref, idx_ref, in_ref, out_ref):
    idxc = idx_ref[0, 0, :].reshape(NPX, 1)
    iot = lax.broadcasted_iota(jnp.int32, (NPX, K), 1)
    onehot = (iot == idxc).astype(jnp.bfloat16)
    g = jnp.dot(onehot, wb_ref[...], preferred_element_type=jnp.float32)
    out_ref[0] = in_ref[0] * g[:, :C] + g[:, C:]


def _tc_affine(wb, idxf3, inp2):
    return pl.pallas_call(
        _tc_affine_body,
        grid=(B, NBLK),
        in_specs=[
            pl.BlockSpec((K, 2 * C), lambda b, j: (0, 0)),
            pl.BlockSpec((1, 1, NPX), lambda b, j: (b, 0, j)),
            pl.BlockSpec((1, NPX, C), lambda b, j: (b, j, 0)),
        ],
        out_specs=pl.BlockSpec((1, NPX, C), lambda b, j: (b, j, 0)),
        out_shape=jax.ShapeDtypeStruct((B, P, C), jnp.float32),
    )(wb, idxf3, inp2)


def kernel(input, mask, weight, bias):
    idx = _sc_argmax(mask)
    wb = jnp.concatenate([weight, bias], axis=1).astype(jnp.bfloat16)
    inp2 = input.transpose(0, 2, 3, 1).reshape(B, P, C)
    out2 = _tc_affine(wb, idx.reshape(B, 1, P), inp2)
    return out2.reshape(B, H, W, C).transpose(0, 3, 1, 2)


# speedup vs baseline: 6.9091x; 1.0087x over previous
import jax
import jax.numpy as jnp
from jax import lax
from jax.experimental import pallas as pl
from jax.experimental.pallas import tpu as pltpu
from jax.experimental.pallas import tpu_sc as plsc

B = 2
C = 256
K = 184
H = 224
W = 224

NW = 32
RB = 8
NPOS = B * (H // RB)
PPB = H // RB
KC = 23
NKC = K // KC
NG = W // 16

P = H * W
NPX = 3584
NBLK = P // NPX


def _sc_argmax_body(bb, mask_hbm, idx_hbm, buf0, buf1, runmax, runidx, sem0, sem1):
    wid = lax.axis_index("s") * 2 + lax.axis_index("c")
    bufs = (buf0, buf1)
    sems = (sem0, sem1)

    def src(i):
        q = i
        h0 = pl.multiple_of((wid % PPB) * RB, RB)
        return wid, mask_hbm.at[bb, pl.ds(q * KC, KC), pl.ds(h0, RB), :]

    def compute(i):
        q = i
        buf = bufs[i % 2]
        qbase = q * KC
        sls = [pl.ds(g * 16, 16) for g in range(NG)]

        def row_body(r, _):
            if q == 0:
                init = tuple(
                    (buf[0, r, sl], jnp.zeros((16,), jnp.int32)) for sl in sls
                )
                lo = 1
            else:
                init = tuple((runmax[r, sl], runidx[r, sl]) for sl in sls)
                lo = 0

            def ch_body(ch, carry):
                chv = jnp.full((16,), qbase + ch, jnp.int32)
                out = []
                for g, sl in enumerate(sls):
                    maxv, idxv = carry[g]
                    v = buf[ch, r, sl]
                    pred = v > maxv
                    out.append((jnp.where(pred, v, maxv),
                                jnp.where(pred, chv, idxv)))
                return tuple(out)

            res = lax.fori_loop(lo, KC, ch_body, init)
            for g, sl in enumerate(sls):
                if q < NKC - 1:
                    runmax[r, sl] = res[g][0]
                runidx[r, sl] = res[g][1]
            return 0

        lax.fori_loop(0, RB, row_body, 0)

        if q == NKC - 1:
            h0 = pl.multiple_of((wid % PPB) * RB, RB)
            pltpu.sync_copy(runidx, idx_hbm.at[0, pl.ds(h0, RB), :])

    NT = NKC

    def valid(i):
        return wid < PPB

    _, s0 = src(0)

    @pl.when(valid(0))
    def _():
        pltpu.make_async_copy(s0, bufs[0], sems[0]).start()
    for i in range(NT):
        if i + 1 < NT:
            _, sn = src(i + 1)

            @pl.when(valid(i + 1))
            def _(sn=sn, i=i):
                pltpu.make_async_copy(sn, bufs[(i + 1) % 2], sems[(i + 1) % 2]).start()

        _, s = src(i)

        @pl.when(valid(i))
        def _(s=s, i=i):
            pltpu.make_async_copy(s, bufs[i % 2], sems[i % 2]).wait()
            compute(i)


def _sc_argmax(mask, bb):
    import functools as _ft
    return pl.kernel(
        _ft.partial(_sc_argmax_body, bb),
        out_type=jax.ShapeDtypeStruct((1, H, W), jnp.int32),
        mesh=plsc.VectorSubcoreMesh(core_axis_name="c", subcore_axis_name="s"),
        scratch_types=[
            pltpu.VMEM((KC, RB, W), jnp.float32),
            pltpu.VMEM((KC, RB, W), jnp.float32),
            pltpu.VMEM((RB, W), jnp.float32),
            pltpu.VMEM((RB, W), jnp.int32),
            pltpu.SemaphoreType.DMA,
            pltpu.SemaphoreType.DMA,
        ],
    )(mask)


def _tc_affine_body(wb_ref, idx_ref, in_ref, out_ref):
    idxc = idx_ref[0, 0, :].reshape(NPX, 1)
    iot = lax.broadcasted_iota(jnp.int32, (NPX, K), 1)
    onehot = (iot == idxc).astype(jnp.bfloat16)
    g = jnp.dot(onehot, wb_ref[...], preferred_element_type=jnp.float32)
    out_ref[0] = in_ref[0] * g[:, :C] + g[:, C:]


def _tc_affine_body_alias(wb_ref, idx_ref, in_ref, carry_ref, out_ref):
    del carry_ref
    _tc_affine_body(wb_ref, idx_ref, in_ref, out_ref)


def _tc_affine(wb, idxf3, inp2, bb, carry=None):
    in_specs = [
        pl.BlockSpec((K, 2 * C), lambda b, j: (0, 0)),
        pl.BlockSpec((1, 1, NPX), lambda b, j: (b, 0, j)),
        pl.BlockSpec((1, NPX, C), lambda b, j, bb=bb: (bb, j, 0)),
    ]
    args = [wb, idxf3, inp2]
    kwargs = {}
    body = _tc_affine_body
    if carry is not None:
        in_specs.append(pl.BlockSpec(memory_space=pl.ANY))
        args.append(carry)
        kwargs["input_output_aliases"] = {3: 0}
        body = _tc_affine_body_alias
    return pl.pallas_call(
        body,
        grid=(1, NBLK),
        in_specs=in_specs,
        out_specs=pl.BlockSpec((1, NPX, C), lambda b, j, bb=bb: (bb, j, 0)),
        out_shape=jax.ShapeDtypeStruct((B, P, C), jnp.float32),
        **kwargs,
    )(*args)


def kernel(input, mask, weight, bias):
    wb = jnp.concatenate([weight, bias], axis=1).astype(jnp.bfloat16)
    inp2 = input.transpose(0, 2, 3, 1).reshape(B, P, C)
    idx0 = _sc_argmax(mask, 0)
    out2 = _tc_affine(wb, idx0.reshape(1, 1, P), inp2, 0)
    idx1 = _sc_argmax(mask, 1)
    out2 = _tc_affine(wb, idx1.reshape(1, 1, P), inp2, 1, carry=out2)
    return out2.reshape(B, H, W, C).transpose(0, 3, 1, 2)
